# Initial kernel scaffold; baseline (speedup 1.0000x reference)
import jax
import jax.numpy as jnp
from jax.experimental import pallas as pl


def kernel(x, enc_Ws, enc_bs, dec_Ws, dec_bs, codebooks):
    raise NotImplementedError("write your pallas kernel here")


# speedup vs baseline: 1.8998x; 1.8998x over previous
import functools

import jax
import jax.numpy as jnp
from jax.experimental import pallas as pl

COMMITMENT_COST = 0.25


def _fused_body(x_ref,
                ew0, ew1, ew2, ew3, eb0, eb1, eb2, eb3,
                dw0, dw1, dw2, dw3, db0, db1, db2, db3,
                cb_ref,
                xr_ref, loss_ref, z_ref, zq_ref, idx_ref,
                *, num_books):
    f32 = jnp.float32

    def dot(a, b):
        return jax.lax.dot(a, b, preferred_element_type=f32)

    h = x_ref[...]
    h = jnp.maximum(dot(h, ew0[...]) + eb0[...], 0.0)
    h = jnp.maximum(dot(h, ew1[...]) + eb1[...], 0.0)
    h = jnp.maximum(dot(h, ew2[...]) + eb2[...], 0.0)
    z = dot(h, ew3[...]) + eb3[...]
    z_ref[...] = z

    residual = z
    z_q = jnp.zeros_like(z)
    loss_sum = f32(0.0)
    idx_cols = []
    blk = z.shape[0]
    for i in range(num_books):
        cb = cb_ref[i]
        r2 = jnp.sum(residual * residual, axis=1, keepdims=True)
        cross = dot(residual, cb.T)
        c2 = jnp.sum(cb * cb, axis=1)
        d = r2 - 2.0 * cross + c2[None, :]
        min_d = jnp.min(d, axis=1, keepdims=True)
        iota = jax.lax.broadcasted_iota(jnp.int32, d.shape, 1)
        idx = jnp.min(jnp.where(d <= min_d, iota, jnp.int32(2**30)), axis=1)
        onehot = (iota == idx[:, None]).astype(f32)
        q = jax.lax.dot(onehot, cb, preferred_element_type=f32,
                        precision=jax.lax.Precision.HIGHEST)
        residual = residual - q
        loss_sum = loss_sum + jnp.sum(residual * residual)
        z_q = z_q + q
        idx_cols.append(idx)

    idx_ref[...] = jnp.stack(idx_cols, axis=1)
    zq_st = z + (z_q - z)
    zq_ref[...] = zq_st

    h = zq_st
    h = jnp.maximum(dot(h, dw0[...]) + db0[...], 0.0)
    h = jnp.maximum(dot(h, dw1[...]) + db1[...], 0.0)
    h = jnp.maximum(dot(h, dw2[...]) + db2[...], 0.0)
    xr_ref[...] = dot(h, dw3[...]) + db3[...]

    @pl.when(pl.program_id(0) == 0)
    def _init():
        loss_ref[...] = jnp.zeros_like(loss_ref)

    loss_ref[...] += loss_sum.reshape(1, 1)


def kernel(x, enc_Ws, enc_bs, dec_Ws, dec_bs, codebooks):
    B, D = x.shape
    num_books, K, C = codebooks.shape
    BLK = min(1024, B)
    grid = B // BLK

    enc_bs2 = [b.reshape(1, -1) for b in enc_bs]
    dec_bs2 = [b.reshape(1, -1) for b in dec_bs]

    full = lambda a: pl.BlockSpec(a.shape, lambda i: (0,) * a.ndim)

    in_specs = ([pl.BlockSpec((BLK, D), lambda i: (i, 0))]
                + [full(w) for w in enc_Ws]
                + [full(b) for b in enc_bs2]
                + [full(w) for w in dec_Ws]
                + [full(b) for b in dec_bs2]
                + [full(codebooks)])

    out_shapes = (
        jax.ShapeDtypeStruct((B, D), jnp.float32),
        jax.ShapeDtypeStruct((1, 1), jnp.float32),
        jax.ShapeDtypeStruct((B, C), jnp.float32),
        jax.ShapeDtypeStruct((B, C), jnp.float32),
        jax.ShapeDtypeStruct((B, num_books), jnp.int32),
    )
    out_specs = (
        pl.BlockSpec((BLK, D), lambda i: (i, 0)),
        pl.BlockSpec((1, 1), lambda i: (0, 0)),
        pl.BlockSpec((BLK, C), lambda i: (i, 0)),
        pl.BlockSpec((BLK, C), lambda i: (i, 0)),
        pl.BlockSpec((BLK, num_books), lambda i: (i, 0)),
    )

    body = functools.partial(_fused_body, num_books=num_books)
    x_recon, loss_sum, z, z_q, idxs = pl.pallas_call(
        body,
        grid=(grid,),
        in_specs=in_specs,
        out_specs=out_specs,
        out_shape=out_shapes,
    )(x, *enc_Ws, *enc_bs2, *dec_Ws, *dec_bs2, codebooks)

    quant_loss = (loss_sum[0, 0] * ((1.0 + COMMITMENT_COST) / (B * C))).astype(jnp.float32)
    return (x_recon, quant_loss, z, z_q, idxs)


# speedup vs baseline: 2.8714x; 1.5114x over previous
import functools

import jax
import jax.numpy as jnp
from jax.experimental import pallas as pl

COMMITMENT_COST = 0.25


def _fused_body(x_ref,
                ew0, ew1, ew2, ew3, eb0, eb1, eb2, eb3,
                dw0, dw1, dw2, dw3, db0, db1, db2, db3,
                cb_ref, cbh_ref, cbm_ref, cbl_ref,
                xr_ref, loss_ref, z_ref, zq_ref, idx_ref,
                *, num_books):
    f32 = jnp.float32

    def dot(a, b):
        return jax.lax.dot(a, b, preferred_element_type=f32)

    h = x_ref[...]
    h = jnp.maximum(dot(h, ew0[...]) + eb0[...], 0.0)
    h = jnp.maximum(dot(h, ew1[...]) + eb1[...], 0.0)
    h = jnp.maximum(dot(h, ew2[...]) + eb2[...], 0.0)
    z = dot(h, ew3[...]) + eb3[...]
    z_ref[...] = z

    residual = z
    z_q = jnp.zeros_like(z)
    loss_sum = f32(0.0)
    idx_cols = []
    blk = z.shape[0]
    for i in range(num_books):
        cb = cb_ref[i]
        r2 = jnp.sum(residual * residual, axis=1, keepdims=True)
        cross = dot(residual, cb.T)
        c2 = jnp.sum(cb * cb, axis=1)
        d = r2 - 2.0 * cross + c2[None, :]
        min_d = jnp.min(d, axis=1, keepdims=True)
        iota = jax.lax.broadcasted_iota(jnp.int32, d.shape, 1)
        idx = jnp.min(jnp.where(d <= min_d, iota, jnp.int32(2**30)), axis=1)
        onehot = (iota == idx[:, None]).astype(jnp.bfloat16)
        q = ((dot(onehot, cbh_ref[i]) + dot(onehot, cbm_ref[i]))
             + dot(onehot, cbl_ref[i]))
        residual = residual - q
        loss_sum = loss_sum + jnp.sum(residual * residual)
        z_q = z_q + q
        idx_cols.append(idx)

    idx_ref[...] = jnp.stack(idx_cols, axis=1)
    zq_st = z + (z_q - z)
    zq_ref[...] = zq_st

    h = zq_st
    h = jnp.maximum(dot(h, dw0[...]) + db0[...], 0.0)
    h = jnp.maximum(dot(h, dw1[...]) + db1[...], 0.0)
    h = jnp.maximum(dot(h, dw2[...]) + db2[...], 0.0)
    xr_ref[...] = dot(h, dw3[...]) + db3[...]

    @pl.when(pl.program_id(0) == 0)
    def _init():
        loss_ref[...] = jnp.zeros_like(loss_ref)

    loss_ref[...] += loss_sum.reshape(1, 1)


def kernel(x, enc_Ws, enc_bs, dec_Ws, dec_bs, codebooks):
    B, D = x.shape
    num_books, K, C = codebooks.shape
    BLK = min(1024, B)
    grid = B // BLK

    enc_bs2 = [b.reshape(1, -1) for b in enc_bs]
    dec_bs2 = [b.reshape(1, -1) for b in dec_bs]

    _mask = jnp.uint32(0xFFFF0000)
    _bc = jax.lax.bitcast_convert_type
    cbh_f = _bc(_bc(codebooks, jnp.uint32) & _mask, jnp.float32)
    _r1 = codebooks - cbh_f
    cbm_f = _bc(_bc(_r1, jnp.uint32) & _mask, jnp.float32)
    cbl_f = _r1 - cbm_f
    cbh = cbh_f.astype(jnp.bfloat16)
    cbm = cbm_f.astype(jnp.bfloat16)
    cbl = cbl_f.astype(jnp.bfloat16)

    full = lambda a: pl.BlockSpec(a.shape, lambda i: (0,) * a.ndim)

    in_specs = ([pl.BlockSpec((BLK, D), lambda i: (i, 0))]
                + [full(w) for w in enc_Ws]
                + [full(b) for b in enc_bs2]
                + [full(w) for w in dec_Ws]
                + [full(b) for b in dec_bs2]
                + [full(codebooks), full(cbh), full(cbm), full(cbl)])

    out_shapes = (
        jax.ShapeDtypeStruct((B, D), jnp.float32),
        jax.ShapeDtypeStruct((1, 1), jnp.float32),
        jax.ShapeDtypeStruct((B, C), jnp.float32),
        jax.ShapeDtypeStruct((B, C), jnp.float32),
        jax.ShapeDtypeStruct((B, num_books), jnp.int32),
    )
    out_specs = (
        pl.BlockSpec((BLK, D), lambda i: (i, 0)),
        pl.BlockSpec((1, 1), lambda i: (0, 0)),
        pl.BlockSpec((BLK, C), lambda i: (i, 0)),
        pl.BlockSpec((BLK, C), lambda i: (i, 0)),
        pl.BlockSpec((BLK, num_books), lambda i: (i, 0)),
    )

    body = functools.partial(_fused_body, num_books=num_books)
    x_recon, loss_sum, z, z_q, idxs = pl.pallas_call(
        body,
        grid=(grid,),
        in_specs=in_specs,
        out_specs=out_specs,
        out_shape=out_shapes,
    )(x, *enc_Ws, *enc_bs2, *dec_Ws, *dec_bs2, codebooks, cbh, cbm, cbl)

    quant_loss = (loss_sum[0, 0] * ((1.0 + COMMITMENT_COST) / (B * C))).astype(jnp.float32)
    return (x_recon, quant_loss, z, z_q, idxs)


# speedup vs baseline: 3.7081x; 1.2914x over previous
import functools

import jax
import jax.numpy as jnp
from jax.experimental import pallas as pl

COMMITMENT_COST = 0.25


def _fused_body(x_ref,
                ew0, ew1, ew2, ew3, eb0, eb1, eb2, eb3,
                dw0, dw1, dw2, dw3, db0, db1, db2, db3,
                cb_ref, cbh_ref, cbm_ref, cbl_ref,
                xr_ref, loss_ref, z_ref, zq_ref, idx_ref,
                *, num_books):
    f32 = jnp.float32

    def dot(a, b):
        return jax.lax.dot(a, b, preferred_element_type=f32)

    h = x_ref[...]
    h = jnp.maximum(dot(h, ew0[...]) + eb0[...], 0.0)
    h = jnp.maximum(dot(h, ew1[...]) + eb1[...], 0.0)
    h = jnp.maximum(dot(h, ew2[...]) + eb2[...], 0.0)
    z = dot(h, ew3[...]) + eb3[...]
    z_ref[...] = z

    blk = z.shape[0]
    H = 2
    sub = blk // H
    residuals = [z[h * sub:(h + 1) * sub] for h in range(H)]
    zqs = [jnp.zeros_like(r) for r in residuals]
    idx_cols = [[] for _ in range(H)]
    loss_parts = []
    for i in range(num_books):
        cb = cb_ref[i]
        cbT = cb.T
        c2 = jnp.sum(cb * cb, axis=1)
        for h in range(H):
            residual = residuals[h]
            r2 = jnp.sum(residual * residual, axis=1, keepdims=True)
            cross = dot(residual, cbT)
            d = r2 - 2.0 * cross + c2[None, :]
            min_d = jnp.min(d, axis=1, keepdims=True)
            iota = jax.lax.broadcasted_iota(jnp.int32, d.shape, 1)
            idx = jnp.min(jnp.where(d <= min_d, iota, jnp.int32(2**30)), axis=1)
            onehot = (iota == idx[:, None]).astype(jnp.bfloat16)
            q = ((dot(onehot, cbh_ref[i]) + dot(onehot, cbm_ref[i]))
                 + dot(onehot, cbl_ref[i]))
            residual = residual - q
            residuals[h] = residual
            loss_parts.append(jnp.sum(residual * residual))
            zqs[h] = zqs[h] + q
            idx_cols[h].append(idx)

    loss_sum = f32(0.0)
    for p in loss_parts:
        loss_sum = loss_sum + p
    idx_ref[...] = jnp.concatenate(
        [jnp.stack(cols, axis=1) for cols in idx_cols], axis=0)
    z_q = jnp.concatenate(zqs, axis=0)
    zq_st = z + (z_q - z)
    zq_ref[...] = zq_st

    h = zq_st
    h = jnp.maximum(dot(h, dw0[...]) + db0[...], 0.0)
    h = jnp.maximum(dot(h, dw1[...]) + db1[...], 0.0)
    h = jnp.maximum(dot(h, dw2[...]) + db2[...], 0.0)
    xr_ref[...] = dot(h, dw3[...]) + db3[...]

    @pl.when(pl.program_id(0) == 0)
    def _init():
        loss_ref[...] = jnp.zeros_like(loss_ref)

    loss_ref[...] += loss_sum.reshape(1, 1)


def kernel(x, enc_Ws, enc_bs, dec_Ws, dec_bs, codebooks):
    B, D = x.shape
    num_books, K, C = codebooks.shape
    BLK = min(2048, B)
    grid = B // BLK

    enc_bs2 = [b.reshape(1, -1) for b in enc_bs]
    dec_bs2 = [b.reshape(1, -1) for b in dec_bs]

    _mask = jnp.uint32(0xFFFF0000)
    _bc = jax.lax.bitcast_convert_type
    cbh_f = _bc(_bc(codebooks, jnp.uint32) & _mask, jnp.float32)
    _r1 = codebooks - cbh_f
    cbm_f = _bc(_bc(_r1, jnp.uint32) & _mask, jnp.float32)
    cbl_f = _r1 - cbm_f
    cbh = cbh_f.astype(jnp.bfloat16)
    cbm = cbm_f.astype(jnp.bfloat16)
    cbl = cbl_f.astype(jnp.bfloat16)

    full = lambda a: pl.BlockSpec(a.shape, lambda i: (0,) * a.ndim)

    in_specs = ([pl.BlockSpec((BLK, D), lambda i: (i, 0))]
                + [full(w) for w in enc_Ws]
                + [full(b) for b in enc_bs2]
                + [full(w) for w in dec_Ws]
                + [full(b) for b in dec_bs2]
                + [full(codebooks), full(cbh), full(cbm), full(cbl)])

    out_shapes = (
        jax.ShapeDtypeStruct((B, D), jnp.float32),
        jax.ShapeDtypeStruct((1, 1), jnp.float32),
        jax.ShapeDtypeStruct((B, C), jnp.float32),
        jax.ShapeDtypeStruct((B, C), jnp.float32),
        jax.ShapeDtypeStruct((B, num_books), jnp.int32),
    )
    out_specs = (
        pl.BlockSpec((BLK, D), lambda i: (i, 0)),
        pl.BlockSpec((1, 1), lambda i: (0, 0)),
        pl.BlockSpec((BLK, C), lambda i: (i, 0)),
        pl.BlockSpec((BLK, C), lambda i: (i, 0)),
        pl.BlockSpec((BLK, num_books), lambda i: (i, 0)),
    )

    body = functools.partial(_fused_body, num_books=num_books)
    x_recon, loss_sum, z, z_q, idxs = pl.pallas_call(
        body,
        grid=(grid,),
        in_specs=in_specs,
        out_specs=out_specs,
        out_shape=out_shapes,
    )(x, *enc_Ws, *enc_bs2, *dec_Ws, *dec_bs2, codebooks, cbh, cbm, cbl)

    quant_loss = (loss_sum[0, 0] * ((1.0 + COMMITMENT_COST) / (B * C))).astype(jnp.float32)
    return (x_recon, quant_loss, z, z_q, idxs)


# speedup vs baseline: 4.0626x; 1.0956x over previous
import functools

import jax
import jax.numpy as jnp
from jax.experimental import pallas as pl

COMMITMENT_COST = 0.25


def _fused_body(x_ref,
                ew0, ew1, ew2, ew3, eb0, eb1, eb2, eb3,
                dw0, dw1, dw2, dw3, db0, db1, db2, db3,
                cb_ref, cbh_ref, cbm_ref, cbl_ref,
                xr_ref, loss_ref, z_ref, zq_ref, idx_ref,
                *, num_books):
    f32 = jnp.float32

    def dot(a, b):
        return jax.lax.dot(a, b, preferred_element_type=f32)

    h = x_ref[...]
    h = jnp.maximum(dot(h, ew0[...]) + eb0[...], 0.0)
    h = jnp.maximum(dot(h, ew1[...]) + eb1[...], 0.0)
    h = jnp.maximum(dot(h, ew2[...]) + eb2[...], 0.0)
    z = dot(h, ew3[...]) + eb3[...]
    z_ref[...] = z

    blk = z.shape[0]
    H = 2
    sub = blk // H
    residuals = [z[h * sub:(h + 1) * sub] for h in range(H)]
    zqs = [jnp.zeros_like(r) for r in residuals]
    idx_cols = [[] for _ in range(H)]
    loss_parts = []
    r2s = [jnp.sum(r * r, axis=1, keepdims=True) for r in residuals]
    iota_f = jax.lax.broadcasted_iota(
        jnp.int32, (sub, cb_ref.shape[1]), 1).astype(f32)
    for i in range(num_books):
        cb = cb_ref[i]
        cbT = cb.T
        c2 = jnp.sum(cb * cb, axis=1)
        for h in range(H):
            residual = residuals[h]
            r2 = r2s[h]
            cross = dot(residual, cbT)
            d = r2 - 2.0 * cross + c2[None, :]
            min_d = jnp.min(d, axis=1, keepdims=True)
            idxf = jnp.min(jnp.where(d <= min_d, iota_f, f32(2.0**30)), axis=1)
            onehot = (iota_f == idxf[:, None]).astype(jnp.bfloat16)
            q = ((dot(onehot, cbh_ref[i]) + dot(onehot, cbm_ref[i]))
                 + dot(onehot, cbl_ref[i]))
            residual = residual - q
            residuals[h] = residual
            r2n = jnp.sum(residual * residual, axis=1, keepdims=True)
            r2s[h] = r2n
            loss_parts.append(jnp.sum(r2n))
            zqs[h] = zqs[h] + q
            idx_cols[h].append(idxf.astype(jnp.int32))

    loss_sum = f32(0.0)
    for p in loss_parts:
        loss_sum = loss_sum + p
    idx_ref[...] = jnp.concatenate(
        [jnp.stack(cols, axis=1) for cols in idx_cols], axis=0)
    z_q = jnp.concatenate(zqs, axis=0)
    zq_st = z + (z_q - z)
    zq_ref[...] = zq_st

    h = zq_st
    h = jnp.maximum(dot(h, dw0[...]) + db0[...], 0.0)
    h = jnp.maximum(dot(h, dw1[...]) + db1[...], 0.0)
    h = jnp.maximum(dot(h, dw2[...]) + db2[...], 0.0)
    xr_ref[...] = dot(h, dw3[...]) + db3[...]

    @pl.when(pl.program_id(0) == 0)
    def _init():
        loss_ref[...] = jnp.zeros_like(loss_ref)

    loss_ref[...] += loss_sum.reshape(1, 1)


def kernel(x, enc_Ws, enc_bs, dec_Ws, dec_bs, codebooks):
    B, D = x.shape
    num_books, K, C = codebooks.shape
    BLK = min(2048, B)
    grid = B // BLK

    enc_bs2 = [b.reshape(1, -1) for b in enc_bs]
    dec_bs2 = [b.reshape(1, -1) for b in dec_bs]

    _mask = jnp.uint32(0xFFFF0000)
    _bc = jax.lax.bitcast_convert_type
    cbh_f = _bc(_bc(codebooks, jnp.uint32) & _mask, jnp.float32)
    _r1 = codebooks - cbh_f
    cbm_f = _bc(_bc(_r1, jnp.uint32) & _mask, jnp.float32)
    cbl_f = _r1 - cbm_f
    cbh = cbh_f.astype(jnp.bfloat16)
    cbm = cbm_f.astype(jnp.bfloat16)
    cbl = cbl_f.astype(jnp.bfloat16)

    full = lambda a: pl.BlockSpec(a.shape, lambda i: (0,) * a.ndim)

    in_specs = ([pl.BlockSpec((BLK, D), lambda i: (i, 0))]
                + [full(w) for w in enc_Ws]
                + [full(b) for b in enc_bs2]
                + [full(w) for w in dec_Ws]
                + [full(b) for b in dec_bs2]
                + [full(codebooks), full(cbh), full(cbm), full(cbl)])

    out_shapes = (
        jax.ShapeDtypeStruct((B, D), jnp.float32),
        jax.ShapeDtypeStruct((1, 1), jnp.float32),
        jax.ShapeDtypeStruct((B, C), jnp.float32),
        jax.ShapeDtypeStruct((B, C), jnp.float32),
        jax.ShapeDtypeStruct((B, num_books), jnp.int32),
    )
    out_specs = (
        pl.BlockSpec((BLK, D), lambda i: (i, 0)),
        pl.BlockSpec((1, 1), lambda i: (0, 0)),
        pl.BlockSpec((BLK, C), lambda i: (i, 0)),
        pl.BlockSpec((BLK, C), lambda i: (i, 0)),
        pl.BlockSpec((BLK, num_books), lambda i: (i, 0)),
    )

    body = functools.partial(_fused_body, num_books=num_books)
    x_recon, loss_sum, z, z_q, idxs = pl.pallas_call(
        body,
        grid=(grid,),
        in_specs=in_specs,
        out_specs=out_specs,
        out_shape=out_shapes,
    )(x, *enc_Ws, *enc_bs2, *dec_Ws, *dec_bs2, codebooks, cbh, cbm, cbl)

    quant_loss = (loss_sum[0, 0] * ((1.0 + COMMITMENT_COST) / (B * C))).astype(jnp.float32)
    return (x_recon, quant_loss, z, z_q, idxs)


# speedup vs baseline: 4.0663x; 1.0009x over previous
import functools

import jax
import jax.numpy as jnp
from jax.experimental import pallas as pl

COMMITMENT_COST = 0.25


def _fused_body(x_ref,
                ew0, ew1, ew2, ew3, eb0, eb1, eb2, eb3,
                dw0, dw1, dw2, dw3, db0, db1, db2, db3,
                cb_ref, cbh_ref, cbm_ref, cbl_ref,
                xr_ref, loss_ref, z_ref, zq_ref, idx_ref,
                *, num_books):
    f32 = jnp.float32

    dd = lambda a, b: jax.lax.dot(a, b, preferred_element_type=f32)

    h = x_ref[...]
    h = jnp.maximum(dd(h, ew0[...]) + eb0[...], 0.0)
    h = jnp.maximum(dd(h, ew1[...]) + eb1[...], 0.0)
    h = jnp.maximum(dd(h, ew2[...]) + eb2[...], 0.0)
    z = dd(h, ew3[...]) + eb3[...]
    z_ref[...] = z

    blk = z.shape[0]
    H = 2
    sub = blk // H
    residuals = [z[h * sub:(h + 1) * sub] for h in range(H)]
    zqs = [jnp.zeros_like(r) for r in residuals]
    idx_cols = [[] for _ in range(H)]
    loss_parts = []
    r2s = [jnp.sum(r * r, axis=1, keepdims=True) for r in residuals]
    iota_f = jax.lax.broadcasted_iota(
        jnp.int32, (sub, cb_ref.shape[1]), 1).astype(f32)
    for i in range(num_books):
        cb = cb_ref[i]
        cbT = cb.T
        c2 = jnp.sum(cb * cb, axis=1)
        for h in range(H):
            residual = residuals[h]
            r2 = r2s[h]
            cross = jax.lax.dot(residual, cbT, preferred_element_type=f32)
            d = r2 - 2.0 * cross + c2[None, :]
            min_d = jnp.min(d, axis=1, keepdims=True)
            idxf = jnp.min(jnp.where(d <= min_d, iota_f, f32(2.0**30)), axis=1)
            onehot = (iota_f == idxf[:, None]).astype(jnp.bfloat16)
            qh = jax.lax.dot(onehot, cbh_ref[i], preferred_element_type=f32)
            qm = jax.lax.dot(onehot, cbm_ref[i], preferred_element_type=f32)
            ql = jax.lax.dot(onehot, cbl_ref[i], preferred_element_type=f32)
            q = (qh + qm) + ql
            residual = residual - q
            residuals[h] = residual
            r2n = jnp.sum(residual * residual, axis=1, keepdims=True)
            r2s[h] = r2n
            loss_parts.append(jnp.sum(r2n))
            zqs[h] = zqs[h] + q
            idx_cols[h].append(idxf.astype(jnp.int32))

    loss_sum = f32(0.0)
    for p in loss_parts:
        loss_sum = loss_sum + p
    idx_ref[...] = jnp.concatenate(
        [jnp.stack(cols, axis=1) for cols in idx_cols], axis=0)
    z_q = jnp.concatenate(zqs, axis=0)
    zq_st = z + (z_q - z)
    zq_ref[...] = zq_st

    h = zq_st
    h = jnp.maximum(jax.lax.dot(h, dw0[...], preferred_element_type=f32) + db0[...], 0.0)
    h = jnp.maximum(jax.lax.dot(h, dw1[...], preferred_element_type=f32) + db1[...], 0.0)
    h = jnp.maximum(jax.lax.dot(h, dw2[...], preferred_element_type=f32) + db2[...], 0.0)
    xr_ref[...] = jax.lax.dot(h, dw3[...], preferred_element_type=f32) + db3[...]

    @pl.when(pl.program_id(0) == 0)
    def _init():
        loss_ref[...] = jnp.zeros_like(loss_ref)

    loss_ref[...] += loss_sum.reshape(1, 1)


def kernel(x, enc_Ws, enc_bs, dec_Ws, dec_bs, codebooks):
    B, D = x.shape
    num_books, K, C = codebooks.shape
    BLK = min(2048, B)
    grid = B // BLK

    enc_bs2 = [b.reshape(1, -1) for b in enc_bs]
    dec_bs2 = [b.reshape(1, -1) for b in dec_bs]

    _mask = jnp.uint32(0xFFFF0000)
    _bc = jax.lax.bitcast_convert_type
    cbh_f = _bc(_bc(codebooks, jnp.uint32) & _mask, jnp.float32)
    _r1 = codebooks - cbh_f
    cbm_f = _bc(_bc(_r1, jnp.uint32) & _mask, jnp.float32)
    cbl_f = _r1 - cbm_f
    cbh = cbh_f.astype(jnp.bfloat16)
    cbm = cbm_f.astype(jnp.bfloat16)
    cbl = cbl_f.astype(jnp.bfloat16)

    full = lambda a: pl.BlockSpec(a.shape, lambda i: (0,) * a.ndim)

    in_specs = ([pl.BlockSpec((BLK, D), lambda i: (i, 0))]
                + [full(w) for w in enc_Ws]
                + [full(b) for b in enc_bs2]
                + [full(w) for w in dec_Ws]
                + [full(b) for b in dec_bs2]
                + [full(codebooks), full(cbh), full(cbm), full(cbl)])

    out_shapes = (
        jax.ShapeDtypeStruct((B, D), jnp.float32),
        jax.ShapeDtypeStruct((1, 1), jnp.float32),
        jax.ShapeDtypeStruct((B, C), jnp.float32),
        jax.ShapeDtypeStruct((B, C), jnp.float32),
        jax.ShapeDtypeStruct((B, num_books), jnp.int32),
    )
    out_specs = (
        pl.BlockSpec((BLK, D), lambda i: (i, 0)),
        pl.BlockSpec((1, 1), lambda i: (0, 0)),
        pl.BlockSpec((BLK, C), lambda i: (i, 0)),
        pl.BlockSpec((BLK, C), lambda i: (i, 0)),
        pl.BlockSpec((BLK, num_books), lambda i: (i, 0)),
    )

    body = functools.partial(_fused_body, num_books=num_books)
    x_recon, loss_sum, z, z_q, idxs = pl.pallas_call(
        body,
        grid=(grid,),
        in_specs=in_specs,
        out_specs=out_specs,
        out_shape=out_shapes,
    )(x, *enc_Ws, *enc_bs2, *dec_Ws, *dec_bs2, codebooks, cbh, cbm, cbl)

    quant_loss = (loss_sum[0, 0] * ((1.0 + COMMITMENT_COST) / (B * C))).astype(jnp.float32)
    return (x_recon, quant_loss, z, z_q, idxs)
